# baseline (speedup 1.0000x reference)
import functools

import jax
import jax.numpy as jnp
from jax import lax
from jax.experimental import pallas as pl
from jax.experimental.pallas import tpu as pltpu, tpu_sc as plsc

_N = 4096
_CE = 4
_CN = 128
_COUT = 128

_BI = 128
_NW = 2
_HW = 16 // _NW

_R_SC = 1280
_R0 = _N - _R_SC
_NI = _R0 // _BI
_NWORK = 32
_RPW = _R_SC // _NWORK


def _tc_body(*refs):
    a_refs = refs[:_NW]
    wp_ref, x_ref, wself_ref, b_ref, dinv_ref, o_ref = refs[_NW:]

    x = a_refs[0][:, 0]
    for w in range(_NW):
        for t in range(_HW):
            if w == 0 and t == 0:
                continue
            x = x + a_refs[w][:, t]
    x = x[:, :4, :] + x[:, 4:, :]
    acc = jnp.sum(x, axis=2)
    msg = (
        jnp.dot(acc, wp_ref[...], preferred_element_type=jnp.float32)
        * dinv_ref[...]
    )
    self_t = jnp.dot(
        x_ref[...], wself_ref[...], preferred_element_type=jnp.float32
    )
    o_ref[...] = jnp.maximum(msg + self_t + b_ref[...], 0.0)


def _epi_body(p_ref, s_ref, x_ref, wself_ref, b_ref, dinv_ref, o_ref):
    msg = (
        jnp.dot(p_ref[...], s_ref[...], preferred_element_type=jnp.float32)
        * dinv_ref[...]
    )
    self_t = jnp.dot(
        x_ref[...], wself_ref[...], preferred_element_type=jnp.float32
    )
    o_ref[...] = jnp.maximum(msg + self_t + b_ref[...], 0.0)


def _sc_reduce(a4):
    mesh = plsc.VectorSubcoreMesh(core_axis_name="c", subcore_axis_name="s")

    @functools.partial(
        pl.kernel,
        mesh=mesh,
        out_type=jax.ShapeDtypeStruct((_R_SC, 64), jnp.float32),
        scratch_types=[
            pltpu.VMEM((16, 8, 128), jnp.float32),
            pltpu.VMEM((64,), jnp.float32),
        ],
    )
    def sc_k(a_hbm, o_hbm, buf, acc_buf):
        cid = lax.axis_index("c")
        sid = lax.axis_index("s")
        wid = sid * 2 + cid
        base = _R0 + wid * _RPW

        def row(r, carry):
            pltpu.sync_copy(a_hbm.at[base + r], buf)
            accs = [jnp.zeros((16,), jnp.float32) for _ in range(4)]
            for a in range(16):
                for s in range(8):
                    for k in range(8):
                        accs[s % 4] = accs[s % 4] + buf[a, s, pl.ds(k * 16, 16)]
            for c in range(4):
                acc_buf[pl.ds(c * 16, 16)] = accs[c]
            pltpu.sync_copy(acc_buf, o_hbm.at[base - _R0 + r])
            return carry

        lax.fori_loop(0, _RPW, row, 0, unroll=False)

    return sc_k(a4)


def kernel(D, A, X, W_pass, b_pass, W_self, b_self):
    A4 = (
        A.reshape(_N, 16, 2, 128, _CE)
        .transpose(0, 1, 2, 4, 3)
        .reshape(_N, 16, 8, 128)
    )
    Wp_T = W_pass.T
    Wself_T = W_self.T
    b = (b_pass + b_self).reshape(1, _COUT)
    Dinv = (1.0 / D).reshape(_N, 1)

    part = _sc_reduce(A4)

    a_specs = [
        pl.BlockSpec((_BI, _HW, 8, 128), lambda i, w=w: (i, w, 0, 0))
        for w in range(_NW)
    ]
    out_main = pl.pallas_call(
        _tc_body,
        grid=(_NI,),
        in_specs=a_specs
        + [
            pl.BlockSpec((_CE, _COUT), lambda i: (0, 0)),
            pl.BlockSpec((_BI, _CN), lambda i: (i, 0)),
            pl.BlockSpec((_CN, _COUT), lambda i: (0, 0)),
            pl.BlockSpec((1, _COUT), lambda i: (0, 0)),
            pl.BlockSpec((_BI, 1), lambda i: (i, 0)),
        ],
        out_specs=pl.BlockSpec((_BI, _COUT), lambda i: (i, 0)),
        out_shape=jax.ShapeDtypeStruct((_R0, _COUT), jnp.float32),
        compiler_params=pltpu.CompilerParams(
            dimension_semantics=("arbitrary",),
        ),
    )(*([A4] * _NW), Wp_T, X[:_R0], Wself_T, b, Dinv[:_R0])

    S = jnp.repeat(Wp_T, 16, axis=0)
    out_tail = pl.pallas_call(
        _epi_body,
        out_shape=jax.ShapeDtypeStruct((_R_SC, _COUT), jnp.float32),
    )(part, S, X[_R0:], Wself_T, b, Dinv[_R0:])

    return jnp.concatenate([out_main, out_tail], axis=0)


# speedup vs baseline: 1.1676x; 1.1676x over previous
import functools

import jax
import jax.numpy as jnp
from jax import lax
from jax.experimental import pallas as pl
from jax.experimental.pallas import tpu as pltpu, tpu_sc as plsc

_N = 4096
_CE = 4
_CN = 128
_COUT = 128

_BI = 128
_NW = 2
_HW = 16 // _NW

_R_SC = 1280
_R0 = _N - _R_SC
_NI = _R0 // _BI
_NWORK = 32
_RPW = _R_SC // _NWORK


def _tc_body(*refs):
    a_refs = refs[:_NW]
    wp_ref, x_ref, wself_ref, b_ref, dinv_ref, o_ref = refs[_NW:]

    x = a_refs[0][:, 0]
    for w in range(_NW):
        for t in range(_HW):
            if w == 0 and t == 0:
                continue
            x = x + a_refs[w][:, t]
    x = x[:, :4, :] + x[:, 4:, :]
    acc = jnp.sum(x, axis=2)
    msg = (
        jnp.dot(acc, wp_ref[...], preferred_element_type=jnp.float32)
        * dinv_ref[...]
    )
    self_t = jnp.dot(
        x_ref[...], wself_ref[...], preferred_element_type=jnp.float32
    )
    o_ref[...] = jnp.maximum(msg + self_t + b_ref[...], 0.0)


def _epi_body(p_ref, s_ref, x_ref, wself_ref, b_ref, dinv_ref, o_ref):
    msg = (
        jnp.dot(p_ref[...], s_ref[...], preferred_element_type=jnp.float32)
        * dinv_ref[...]
    )
    self_t = jnp.dot(
        x_ref[...], wself_ref[...], preferred_element_type=jnp.float32
    )
    o_ref[...] = jnp.maximum(msg + self_t + b_ref[...], 0.0)


def _sc_reduce(a4):
    mesh = plsc.VectorSubcoreMesh(core_axis_name="c", subcore_axis_name="s")

    @functools.partial(
        pl.kernel,
        mesh=mesh,
        out_type=jax.ShapeDtypeStruct((_R_SC, 64), jnp.float32),
        scratch_types=[
            pltpu.VMEM((2, 16, 8, 128), jnp.float32),
            pltpu.VMEM((2, 64), jnp.float32),
            pltpu.SemaphoreType.DMA((2,)),
        ],
    )
    def sc_k(a_hbm, o_hbm, buf, acc_buf, sems):
        cid = lax.axis_index("c")
        sid = lax.axis_index("s")
        wid = sid * 2 + cid
        base = _R0 + wid * _RPW

        pltpu.async_copy(a_hbm.at[base], buf.at[0], sems.at[0]).start()

        def row(r, b):
            pltpu.async_copy(a_hbm.at[base + r], buf.at[b], sems.at[b]).wait()

            @pl.when(r + 1 < _RPW)
            def _():
                pltpu.async_copy(
                    a_hbm.at[base + r + 1], buf.at[1 - b], sems.at[1 - b]
                ).start()

            accs = [jnp.zeros((16,), jnp.float32) for _ in range(8)]
            for a in range(16):
                for s in range(8):
                    for k in range(8):
                        accs[(s % 4) * 2 + (k % 2)] = (
                            accs[(s % 4) * 2 + (k % 2)]
                            + buf[b, a, s, pl.ds(k * 16, 16)]
                        )
            for c in range(4):
                acc_buf[b, pl.ds(c * 16, 16)] = accs[2 * c] + accs[2 * c + 1]
            pltpu.sync_copy(acc_buf.at[b], o_hbm.at[base - _R0 + r])

        def rr(q, carry):
            row(2 * q, 0)
            row(2 * q + 1, 1)
            return carry

        lax.fori_loop(0, _RPW // 2, rr, 0, unroll=False)

    return sc_k(a4)


def kernel(D, A, X, W_pass, b_pass, W_self, b_self):
    A4 = (
        A.reshape(_N, 16, 2, 128, _CE)
        .transpose(0, 1, 2, 4, 3)
        .reshape(_N, 16, 8, 128)
    )
    Wp_T = W_pass.T
    Wself_T = W_self.T
    b = (b_pass + b_self).reshape(1, _COUT)
    Dinv = (1.0 / D).reshape(_N, 1)

    part = _sc_reduce(A4)

    a_specs = [
        pl.BlockSpec((_BI, _HW, 8, 128), lambda i, w=w: (i, w, 0, 0))
        for w in range(_NW)
    ]
    out_main = pl.pallas_call(
        _tc_body,
        grid=(_NI,),
        in_specs=a_specs
        + [
            pl.BlockSpec((_CE, _COUT), lambda i: (0, 0)),
            pl.BlockSpec((_BI, _CN), lambda i: (i, 0)),
            pl.BlockSpec((_CN, _COUT), lambda i: (0, 0)),
            pl.BlockSpec((1, _COUT), lambda i: (0, 0)),
            pl.BlockSpec((_BI, 1), lambda i: (i, 0)),
        ],
        out_specs=pl.BlockSpec((_BI, _COUT), lambda i: (i, 0)),
        out_shape=jax.ShapeDtypeStruct((_R0, _COUT), jnp.float32),
        compiler_params=pltpu.CompilerParams(
            dimension_semantics=("arbitrary",),
        ),
    )(*([A4] * _NW), Wp_T, X[:_R0], Wself_T, b, Dinv[:_R0])

    S = jnp.repeat(Wp_T, 16, axis=0)
    out_tail = pl.pallas_call(
        _epi_body,
        out_shape=jax.ShapeDtypeStruct((_R_SC, _COUT), jnp.float32),
    )(part, S, X[_R0:], Wself_T, b, Dinv[_R0:])

    return jnp.concatenate([out_main, out_tail], axis=0)


# speedup vs baseline: 2.0098x; 1.7212x over previous
import functools

import jax
import jax.numpy as jnp
from jax import lax
from jax.experimental import pallas as pl
from jax.experimental.pallas import tpu as pltpu, tpu_sc as plsc

_N = 4096
_CE = 4
_CN = 128
_COUT = 128

_BI = 128
_NW = 2
_HW = 16 // _NW

_R_SC = 384
_R0 = _N - _R_SC
_NI = _R0 // _BI
_NWORK = 32
_RPW = _R_SC // _NWORK


def _tc_body(*refs):
    a_refs = refs[:_NW]
    wp_ref, x_ref, wself_ref, b_ref, dinv_ref, o_ref = refs[_NW:]

    x = a_refs[0][:, 0]
    for w in range(_NW):
        for t in range(_HW):
            if w == 0 and t == 0:
                continue
            x = x + a_refs[w][:, t]
    x = x[:, :4, :] + x[:, 4:, :]
    acc = jnp.sum(x, axis=2)
    msg = (
        jnp.dot(acc, wp_ref[...], preferred_element_type=jnp.float32)
        * dinv_ref[...]
    )
    self_t = jnp.dot(
        x_ref[...], wself_ref[...], preferred_element_type=jnp.float32
    )
    o_ref[...] = jnp.maximum(msg + self_t + b_ref[...], 0.0)


def _epi_body(p_ref, s_ref, x_ref, wself_ref, b_ref, dinv_ref, o_ref):
    msg = (
        jnp.dot(p_ref[...], s_ref[...], preferred_element_type=jnp.float32)
        * dinv_ref[...]
    )
    self_t = jnp.dot(
        x_ref[...], wself_ref[...], preferred_element_type=jnp.float32
    )
    o_ref[...] = jnp.maximum(msg + self_t + b_ref[...], 0.0)


def _sc_reduce(a4):
    mesh = plsc.VectorSubcoreMesh(core_axis_name="c", subcore_axis_name="s")

    @functools.partial(
        pl.kernel,
        mesh=mesh,
        out_type=jax.ShapeDtypeStruct((_R_SC, 64), jnp.float32),
        scratch_types=[
            pltpu.VMEM((16, 8, 128), jnp.float32),
            pltpu.VMEM((64,), jnp.float32),
        ],
    )
    def sc_k(a_hbm, o_hbm, buf, acc_buf):
        cid = lax.axis_index("c")
        sid = lax.axis_index("s")
        wid = sid * 2 + cid
        base = _R0 + wid * _RPW

        def row(r, carry):
            pltpu.sync_copy(a_hbm.at[base + r], buf)
            accs = [jnp.zeros((16,), jnp.float32) for _ in range(8)]
            for a in range(16):
                for s in range(8):
                    for k in range(8):
                        accs[(s % 4) * 2 + (k % 2)] = (
                            accs[(s % 4) * 2 + (k % 2)]
                            + buf[a, s, pl.ds(k * 16, 16)]
                        )
            for c in range(4):
                acc_buf[pl.ds(c * 16, 16)] = accs[2 * c] + accs[2 * c + 1]
            pltpu.sync_copy(acc_buf, o_hbm.at[base - _R0 + r])
            return carry

        lax.fori_loop(0, _RPW, row, 0, unroll=False)

    return sc_k(a4)


def kernel(D, A, X, W_pass, b_pass, W_self, b_self):
    A4 = (
        A.reshape(_N, 16, 2, 128, _CE)
        .transpose(0, 1, 2, 4, 3)
        .reshape(_N, 16, 8, 128)
    )
    Wp_T = W_pass.T
    Wself_T = W_self.T
    b = (b_pass + b_self).reshape(1, _COUT)
    Dinv = (1.0 / D).reshape(_N, 1)

    part = _sc_reduce(A4)

    a_specs = [
        pl.BlockSpec((_BI, _HW, 8, 128), lambda i, w=w: (i, w, 0, 0))
        for w in range(_NW)
    ]
    out_main = pl.pallas_call(
        _tc_body,
        grid=(_NI,),
        in_specs=a_specs
        + [
            pl.BlockSpec((_CE, _COUT), lambda i: (0, 0)),
            pl.BlockSpec((_BI, _CN), lambda i: (i, 0)),
            pl.BlockSpec((_CN, _COUT), lambda i: (0, 0)),
            pl.BlockSpec((1, _COUT), lambda i: (0, 0)),
            pl.BlockSpec((_BI, 1), lambda i: (i, 0)),
        ],
        out_specs=pl.BlockSpec((_BI, _COUT), lambda i: (i, 0)),
        out_shape=jax.ShapeDtypeStruct((_R0, _COUT), jnp.float32),
        compiler_params=pltpu.CompilerParams(
            dimension_semantics=("arbitrary",),
        ),
    )(*([A4] * _NW), Wp_T, X[:_R0], Wself_T, b, Dinv[:_R0])

    S = jnp.repeat(Wp_T, 16, axis=0)
    out_tail = pl.pallas_call(
        _epi_body,
        out_shape=jax.ShapeDtypeStruct((_R_SC, _COUT), jnp.float32),
    )(part, S, X[_R0:], Wself_T, b, Dinv[_R0:])

    return jnp.concatenate([out_main, out_tail], axis=0)


# speedup vs baseline: 2.5412x; 1.2644x over previous
import jax
import jax.numpy as jnp
from jax.experimental import pallas as pl
from jax.experimental.pallas import tpu as pltpu

_N = 4096
_CE = 4
_CN = 128
_COUT = 128

_BI = 128
_NI = _N // _BI
_NW = 4
_HW = 16 // _NW


def _body(*refs):
    a_refs = refs[:_NW]
    wp_ref, x_ref, wself_ref, b_ref, dinv_ref, o_ref = refs[_NW:]

    x = a_refs[0][:, 0]
    for w in range(_NW):
        for t in range(_HW):
            if w == 0 and t == 0:
                continue
            x = x + a_refs[w][:, t]
    x = x[:, :4, :] + x[:, 4:, :]
    acc = jnp.sum(x, axis=2)
    msg = (
        jnp.dot(acc, wp_ref[...], preferred_element_type=jnp.float32)
        * dinv_ref[...]
    )
    self_t = jnp.dot(
        x_ref[...], wself_ref[...], preferred_element_type=jnp.float32
    )
    o_ref[...] = jnp.maximum(msg + self_t + b_ref[...], 0.0)


def kernel(D, A, X, W_pass, b_pass, W_self, b_self):
    A4 = (
        A.reshape(_N, 16, 2, 128, _CE)
        .transpose(0, 1, 2, 4, 3)
        .reshape(_N, 16, 8, 128)
    )
    Wp_T = W_pass.T
    Wself_T = W_self.T
    b = (b_pass + b_self).reshape(1, _COUT)
    Dinv = (1.0 / D).reshape(_N, 1)

    a_specs = [
        pl.BlockSpec((_BI, _HW, 8, 128), lambda i, w=w: (i, w, 0, 0))
        for w in range(_NW)
    ]
    out = pl.pallas_call(
        _body,
        grid=(_NI,),
        in_specs=a_specs
        + [
            pl.BlockSpec((_CE, _COUT), lambda i: (0, 0)),
            pl.BlockSpec((_BI, _CN), lambda i: (i, 0)),
            pl.BlockSpec((_CN, _COUT), lambda i: (0, 0)),
            pl.BlockSpec((1, _COUT), lambda i: (0, 0)),
            pl.BlockSpec((_BI, 1), lambda i: (i, 0)),
        ],
        out_specs=pl.BlockSpec((_BI, _COUT), lambda i: (i, 0)),
        out_shape=jax.ShapeDtypeStruct((_N, _COUT), jnp.float32),
        compiler_params=pltpu.CompilerParams(
            dimension_semantics=("arbitrary",),
        ),
    )(*([A4] * _NW), Wp_T, X, Wself_T, b, Dinv)
    return out
